# Initial kernel scaffold; baseline (speedup 1.0000x reference)
import jax
import jax.numpy as jnp
from jax.experimental import pallas as pl


def kernel(out, shorty, topk_C_vals, topk_C_inds, y_inds):
    raise NotImplementedError("write your pallas kernel here")


# speedup vs baseline: 117.6218x; 117.6218x over previous
import functools

import jax
import jax.numpy as jnp
from jax import lax
from jax.experimental import pallas as pl
from jax.experimental.pallas import tpu as pltpu
from jax.experimental.pallas import tpu_sc as plsc

GAMMA_ = 0.05
LOG2_ = 0.6931471805599453

_NC, _NS, _LANES = 2, 16, 16
_NW = _NC * _NS

_B, _S, _K, _LY = 4096, 500, 200, 50
_SP_, _KP_, _YP_ = 512, 208, 64
_PADQ = 100001
_PADY = 100000
_MASKN = 100352

_RPW = _B // _NW
_CH = 8
_NCHUNK = _RPW // _CH


def _tc_body(out_ref, vals_ref, a_ref, sp_ref):
    step = pl.program_id(0)
    x = out_ref[...]
    a_part = jnp.sum(jnp.maximum(x, 0.0) + jnp.log(1.0 + jnp.exp(-jnp.abs(x))))

    @pl.when(step == 0)
    def _():
        a_ref[0, 0] = 0.0

    a_ref[0, 0] += a_part
    v = vals_ref[...]
    sp_ref[...] = jnp.maximum(v, 0.0) - v + jnp.log(1.0 + jnp.exp(-jnp.abs(v)))


def _tc_dense(out, vals_p):
    blk = 512
    grid = (_B // blk,)
    return pl.pallas_call(
        _tc_body,
        grid=grid,
        in_specs=[
            pl.BlockSpec((blk, _S), lambda i: (i, 0)),
            pl.BlockSpec((blk, _KP_), lambda i: (i, 0)),
        ],
        out_specs=[
            pl.BlockSpec(memory_space=pltpu.SMEM),
            pl.BlockSpec((blk, _KP_), lambda i: (i, 0)),
        ],
        out_shape=[
            jax.ShapeDtypeStruct((1, 1), jnp.float32),
            jax.ShapeDtypeStruct((_B, _KP_), jnp.float32),
        ],
    )(out, vals_p)


def _sc_body(sh_hbm, y_hbm, tk_hbm, o_hbm, sp_hbm,
             s1_out, h_out, cs_out, cm_out,
             mask_v, sh_v, y_v, tk_v, o_v, sp_v,
             r1_v, r2_v, r3_v, r4_v):
    wid = lax.axis_index("s") * _NC + lax.axis_index("c")
    row0 = wid * _RPW
    neg1 = jnp.full((_LANES,), -1, jnp.int32)

    def init_body(i, _):
        for t in range(16):
            mask_v[pl.ds(i * 256 + t * 16, 16)] = neg1
        return 0

    lax.fori_loop(0, _MASKN // 256, init_body, 0)

    zf = jnp.zeros((_LANES,), jnp.float32)
    zi = jnp.zeros((_LANES,), jnp.int32)
    onei = jnp.full((_LANES,), 1, jnp.int32)

    def chunk_body(ci, carry):
        s1, h, csum, cmax = carry
        base = row0 + ci * _CH
        pltpu.sync_copy(sh_hbm.at[pl.ds(base * _SP_, _CH * _SP_)], sh_v)
        pltpu.sync_copy(y_hbm.at[pl.ds(base * _YP_, _CH * _YP_)], y_v)
        pltpu.sync_copy(tk_hbm.at[pl.ds(base * _KP_, _CH * _KP_)], tk_v)
        pltpu.sync_copy(o_hbm.at[pl.ds(base * _SP_, _CH * _SP_)], o_v)
        pltpu.sync_copy(sp_hbm.at[pl.ds(base * _KP_, _CH * _KP_)], sp_v)
        for r in range(_CH):
            tag = jnp.full((_LANES,), base + r, jnp.int32)
            for t in range(_YP_ // 16):
                yv = y_v[pl.ds(r * _YP_ + t * 16, 16)]
                plsc.store_scatter(mask_v, [yv], tag)
            for j in range(_SP_ // 16):
                q = sh_v[pl.ds(r * _SP_ + j * 16, 16)]
                m = plsc.load_gather(mask_v, [q])
                o = o_v[pl.ds(r * _SP_ + j * 16, 16)]
                s1 = s1 + jnp.where(m == tag, o, zf)
            rowcnt = zi
            for j in range(_KP_ // 16):
                q = tk_v[pl.ds(r * _KP_ + j * 16, 16)]
                m = plsc.load_gather(mask_v, [q])
                spv = sp_v[pl.ds(r * _KP_ + j * 16, 16)]
                hit = m == tag
                h = h + jnp.where(hit, spv, zf)
                rowcnt = rowcnt + jnp.where(hit, onei, zi)
            cmax = jnp.maximum(cmax, jnp.sum(rowcnt))
            csum = csum + rowcnt
        return (s1, h, csum, cmax)

    s1, h, csum, cmax = lax.fori_loop(
        0, _NCHUNK, chunk_body,
        (zf, zf, zi, jnp.int32(0)))

    r1_v[...] = s1
    r2_v[...] = h
    r3_v[...] = csum
    r4_v[...] = jnp.full((_LANES,), cmax, jnp.int32)
    pltpu.sync_copy(r1_v, s1_out.at[pl.ds(wid * _LANES, _LANES)])
    pltpu.sync_copy(r2_v, h_out.at[pl.ds(wid * _LANES, _LANES)])
    pltpu.sync_copy(r3_v, cs_out.at[pl.ds(wid * _LANES, _LANES)])
    pltpu.sync_copy(r4_v, cm_out.at[pl.ds(wid * _LANES, _LANES)])


def _sc_membership(sh_p, y_p, tk_p, out_p, sp_p):
    mesh = plsc.VectorSubcoreMesh(core_axis_name="c", subcore_axis_name="s")
    f = pl.kernel(
        _sc_body,
        out_type=[
            jax.ShapeDtypeStruct((_NW * _LANES,), jnp.float32),
            jax.ShapeDtypeStruct((_NW * _LANES,), jnp.float32),
            jax.ShapeDtypeStruct((_NW * _LANES,), jnp.int32),
            jax.ShapeDtypeStruct((_NW * _LANES,), jnp.int32),
        ],
        mesh=mesh,
        compiler_params=pltpu.CompilerParams(needs_layout_passes=False),
        scratch_types=[
            pltpu.VMEM((_MASKN,), jnp.int32),
            pltpu.VMEM((_CH * _SP_,), jnp.int32),
            pltpu.VMEM((_CH * _YP_,), jnp.int32),
            pltpu.VMEM((_CH * _KP_,), jnp.int32),
            pltpu.VMEM((_CH * _SP_,), jnp.float32),
            pltpu.VMEM((_CH * _KP_,), jnp.float32),
            pltpu.VMEM((_LANES,), jnp.float32),
            pltpu.VMEM((_LANES,), jnp.float32),
            pltpu.VMEM((_LANES,), jnp.int32),
            pltpu.VMEM((_LANES,), jnp.int32),
        ],
    )
    return f(sh_p, y_p, tk_p, out_p, sp_p)


def kernel(out, shorty, topk_C_vals, topk_C_inds, y_inds):
    B, S = out.shape
    K = topk_C_vals.shape[1]
    sh_p = jnp.pad(shorty.astype(jnp.int32), ((0, 0), (0, _SP_ - S)),
                   constant_values=_PADQ).reshape(-1)
    tk_p = jnp.pad(topk_C_inds.astype(jnp.int32), ((0, 0), (0, _KP_ - K)),
                   constant_values=_PADQ).reshape(-1)
    y_p = jnp.pad(y_inds.astype(jnp.int32), ((0, 0), (0, _YP_ - _LY)),
                  constant_values=_PADY).reshape(-1)
    out_p = jnp.pad(out, ((0, 0), (0, _SP_ - S))).reshape(-1)
    vals_p = jnp.pad(topk_C_vals, ((0, 0), (0, _KP_ - K)))

    a_arr, sp_p = _tc_dense(out, vals_p)
    s1_w, h_w, cs_w, cm_w = _sc_membership(sh_p, y_p, tk_p, out_p,
                                           sp_p.reshape(-1))

    A = a_arr[0, 0]
    S1 = jnp.sum(s1_w)
    H = jnp.sum(h_w)
    csum = jnp.sum(cs_w).astype(jnp.float32)
    k = jnp.maximum(jnp.max(cm_w), 1).astype(jnp.float32)
    loss_precision = (A - S1) / jnp.float32(B * S)
    n = jnp.float32(B) * k
    loss_recall = (H + (n - csum) * jnp.float32(LOG2_)) / n
    return loss_precision + jnp.float32(GAMMA_) * loss_recall


# speedup vs baseline: 148.7184x; 1.2644x over previous
import functools

import jax
import jax.numpy as jnp
from jax import lax
from jax.experimental import pallas as pl
from jax.experimental.pallas import tpu as pltpu
from jax.experimental.pallas import tpu_sc as plsc

GAMMA_ = 0.05
LOG2_ = 0.6931471805599453

_NC, _NS, _LANES = 2, 16, 16
_NW = _NC * _NS

_B, _S, _K, _LY = 4096, 500, 200, 50
_SP_, _KP_, _YP_ = 512, 208, 64
_PADQ = 100001
_PADY = 100000
_MASKN = 100352

_RPW = _B // _NW
_CH = 4
_NCHUNK = _RPW // _CH


def _tc_body(out_ref, vals_ref, a_ref, sp_ref):
    step = pl.program_id(0)
    x = out_ref[...]
    a_part = jnp.sum(jnp.maximum(x, 0.0) + jnp.log(1.0 + jnp.exp(-jnp.abs(x))))

    @pl.when(step == 0)
    def _():
        a_ref[0, 0] = 0.0

    a_ref[0, 0] += a_part
    v = vals_ref[...]
    sp_ref[...] = jnp.maximum(v, 0.0) - v + jnp.log(1.0 + jnp.exp(-jnp.abs(v)))


def _tc_dense(out, vals_p):
    blk = 512
    grid = (_B // blk,)
    return pl.pallas_call(
        _tc_body,
        grid=grid,
        in_specs=[
            pl.BlockSpec((blk, _S), lambda i: (i, 0)),
            pl.BlockSpec((blk, _KP_), lambda i: (i, 0)),
        ],
        out_specs=[
            pl.BlockSpec(memory_space=pltpu.SMEM),
            pl.BlockSpec((blk, _KP_), lambda i: (i, 0)),
        ],
        out_shape=[
            jax.ShapeDtypeStruct((1, 1), jnp.float32),
            jax.ShapeDtypeStruct((_B, _KP_), jnp.float32),
        ],
    )(out, vals_p)


def _sc_body(sh_hbm, y_hbm, tk_hbm, o_hbm, sp_hbm,
             s1_out, h_out, cs_out, cm_out,
             mask_v, sh_v0, y_v0, tk_v0, o_v0, sp_v0,
             sh_v1, y_v1, tk_v1, o_v1, sp_v1,
             r1_v, r2_v, r3_v, r4_v, sem0, sem1):
    wid = lax.axis_index("s") * _NC + lax.axis_index("c")
    row0 = wid * _RPW
    neg1 = jnp.full((_LANES,), -1, jnp.int32)
    bufs = ((sh_v0, y_v0, tk_v0, o_v0, sp_v0),
            (sh_v1, y_v1, tk_v1, o_v1, sp_v1))
    sems = (sem0, sem1)

    def _copies(ci, slot):
        base = row0 + ci * _CH
        sh_v, y_v, tk_v, o_v, sp_v = bufs[slot]
        return (
            (sh_hbm.at[pl.ds(base * _SP_, _CH * _SP_)], sh_v),
            (y_hbm.at[pl.ds(base * _YP_, _CH * _YP_)], y_v),
            (tk_hbm.at[pl.ds(base * _KP_, _CH * _KP_)], tk_v),
            (o_hbm.at[pl.ds(base * _SP_, _CH * _SP_)], o_v),
            (sp_hbm.at[pl.ds(base * _KP_, _CH * _KP_)], sp_v),
        )

    def _issue(ci, slot):
        for src, dst in _copies(ci, slot):
            pltpu.async_copy(src, dst, sems[slot])

    def _wait(ci, slot):
        for src, dst in _copies(ci, slot):
            pltpu.make_async_copy(src, dst, sems[slot]).wait()

    _issue(0, 0)
    _issue(1, 1)

    def init_body(i, _):
        for t in range(16):
            mask_v[pl.ds(i * 256 + t * 16, 16)] = neg1
        return 0

    lax.fori_loop(0, _MASKN // 256, init_body, 0)

    zf = jnp.zeros((_LANES,), jnp.float32)
    zi = jnp.zeros((_LANES,), jnp.int32)
    onei = jnp.full((_LANES,), 1, jnp.int32)
    last = _NCHUNK - 1

    def _compute(ci, slot, carry):
        s1, h, csum, cmax = carry
        base = row0 + ci * _CH
        sh_v, y_v, tk_v, o_v, sp_v = bufs[slot]
        for r in range(_CH):
            tag = jnp.full((_LANES,), base + r, jnp.int32)
            for t in range(_YP_ // 16):
                yv = y_v[pl.ds(r * _YP_ + t * 16, 16)]
                plsc.store_scatter(mask_v, [yv], tag)
            for j in range(_SP_ // 16):
                q = sh_v[pl.ds(r * _SP_ + j * 16, 16)]
                m = plsc.load_gather(mask_v, [q])
                o = o_v[pl.ds(r * _SP_ + j * 16, 16)]
                s1 = s1 + jnp.where(m == tag, o, zf)
            rowcnt = zi
            for j in range(_KP_ // 16):
                q = tk_v[pl.ds(r * _KP_ + j * 16, 16)]
                m = plsc.load_gather(mask_v, [q])
                spv = sp_v[pl.ds(r * _KP_ + j * 16, 16)]
                hit = m == tag
                h = h + jnp.where(hit, spv, zf)
                rowcnt = rowcnt + jnp.where(hit, onei, zi)
            cmax = jnp.maximum(cmax, jnp.sum(rowcnt))
            csum = csum + rowcnt
        return (s1, h, csum, cmax)

    def pair_body(p, carry):
        c0 = p * 2
        c1 = c0 + 1
        _wait(c0, 0)
        carry = _compute(c0, 0, carry)
        _issue(jnp.minimum(c0 + 2, last), 0)
        _wait(c1, 1)
        carry = _compute(c1, 1, carry)
        _issue(jnp.minimum(c1 + 2, last), 1)
        return carry

    s1, h, csum, cmax = lax.fori_loop(
        0, _NCHUNK // 2, pair_body,
        (zf, zf, zi, jnp.int32(0)))
    _wait(last, 0)
    _wait(last, 1)

    r1_v[...] = s1
    r2_v[...] = h
    r3_v[...] = csum
    r4_v[...] = jnp.full((_LANES,), cmax, jnp.int32)
    pltpu.sync_copy(r1_v, s1_out.at[pl.ds(wid * _LANES, _LANES)])
    pltpu.sync_copy(r2_v, h_out.at[pl.ds(wid * _LANES, _LANES)])
    pltpu.sync_copy(r3_v, cs_out.at[pl.ds(wid * _LANES, _LANES)])
    pltpu.sync_copy(r4_v, cm_out.at[pl.ds(wid * _LANES, _LANES)])


def _sc_membership(sh_p, y_p, tk_p, out_p, sp_p):
    mesh = plsc.VectorSubcoreMesh(core_axis_name="c", subcore_axis_name="s")
    f = pl.kernel(
        _sc_body,
        out_type=[
            jax.ShapeDtypeStruct((_NW * _LANES,), jnp.float32),
            jax.ShapeDtypeStruct((_NW * _LANES,), jnp.float32),
            jax.ShapeDtypeStruct((_NW * _LANES,), jnp.int32),
            jax.ShapeDtypeStruct((_NW * _LANES,), jnp.int32),
        ],
        mesh=mesh,
        compiler_params=pltpu.CompilerParams(needs_layout_passes=False),
        scratch_types=[
            pltpu.VMEM((_MASKN,), jnp.int32),
            pltpu.VMEM((_CH * _SP_,), jnp.int32),
            pltpu.VMEM((_CH * _YP_,), jnp.int32),
            pltpu.VMEM((_CH * _KP_,), jnp.int32),
            pltpu.VMEM((_CH * _SP_,), jnp.float32),
            pltpu.VMEM((_CH * _KP_,), jnp.float32),
            pltpu.VMEM((_CH * _SP_,), jnp.int32),
            pltpu.VMEM((_CH * _YP_,), jnp.int32),
            pltpu.VMEM((_CH * _KP_,), jnp.int32),
            pltpu.VMEM((_CH * _SP_,), jnp.float32),
            pltpu.VMEM((_CH * _KP_,), jnp.float32),
            pltpu.VMEM((_LANES,), jnp.float32),
            pltpu.VMEM((_LANES,), jnp.float32),
            pltpu.VMEM((_LANES,), jnp.int32),
            pltpu.VMEM((_LANES,), jnp.int32),
            pltpu.SemaphoreType.DMA,
            pltpu.SemaphoreType.DMA,
        ],
    )
    return f(sh_p, y_p, tk_p, out_p, sp_p)


def kernel(out, shorty, topk_C_vals, topk_C_inds, y_inds):
    B, S = out.shape
    K = topk_C_vals.shape[1]
    sh_p = jnp.pad(shorty.astype(jnp.int32), ((0, 0), (0, _SP_ - S)),
                   constant_values=_PADQ).reshape(-1)
    tk_p = jnp.pad(topk_C_inds.astype(jnp.int32), ((0, 0), (0, _KP_ - K)),
                   constant_values=_PADQ).reshape(-1)
    y_p = jnp.pad(y_inds.astype(jnp.int32), ((0, 0), (0, _YP_ - _LY)),
                  constant_values=_PADY).reshape(-1)
    out_p = jnp.pad(out, ((0, 0), (0, _SP_ - S))).reshape(-1)
    vals_p = jnp.pad(topk_C_vals, ((0, 0), (0, _KP_ - K)))

    a_arr, sp_p = _tc_dense(out, vals_p)
    s1_w, h_w, cs_w, cm_w = _sc_membership(sh_p, y_p, tk_p, out_p,
                                           sp_p.reshape(-1))

    A = a_arr[0, 0]
    S1 = jnp.sum(s1_w)
    H = jnp.sum(h_w)
    csum = jnp.sum(cs_w).astype(jnp.float32)
    k = jnp.maximum(jnp.max(cm_w), 1).astype(jnp.float32)
    loss_precision = (A - S1) / jnp.float32(B * S)
    n = jnp.float32(B) * k
    loss_recall = (H + (n - csum) * jnp.float32(LOG2_)) / n
    return loss_precision + jnp.float32(GAMMA_) * loss_recall


# speedup vs baseline: 159.4194x; 1.0720x over previous
import functools

import jax
import jax.numpy as jnp
from jax import lax
from jax.experimental import pallas as pl
from jax.experimental.pallas import tpu as pltpu
from jax.experimental.pallas import tpu_sc as plsc

GAMMA_ = 0.05
LOG2_ = 0.6931471805599453

_NC, _NS, _LANES = 2, 16, 16
_NW = _NC * _NS

_B, _S, _K, _LY = 4096, 500, 200, 50
_PADQ = 100001
_PADY = 100000
_MASKN = 100352

_RPW = _B // _NW
_CH = 4
_NCHUNK = _RPW // _CH


def _tc_body(out_ref, vals_ref, a_ref, sp_ref):
    step = pl.program_id(0)
    x = out_ref[...]
    a_part = jnp.sum(jnp.maximum(x, 0.0) + jnp.log(1.0 + jnp.exp(-jnp.abs(x))))

    @pl.when(step == 0)
    def _():
        a_ref[0, 0] = 0.0

    a_ref[0, 0] += a_part
    v = vals_ref[...]
    sp_ref[...] = jnp.maximum(v, 0.0) - v + jnp.log(1.0 + jnp.exp(-jnp.abs(v)))


def _tc_dense(out, vals):
    blk = 512
    grid = (_B // blk,)
    return pl.pallas_call(
        _tc_body,
        grid=grid,
        in_specs=[
            pl.BlockSpec((blk, _S), lambda i: (i, 0)),
            pl.BlockSpec((blk, _K), lambda i: (i, 0)),
        ],
        out_specs=[
            pl.BlockSpec(memory_space=pltpu.SMEM),
            pl.BlockSpec((blk, _K), lambda i: (i, 0)),
        ],
        out_shape=[
            jax.ShapeDtypeStruct((1, 1), jnp.float32),
            jax.ShapeDtypeStruct((_B, _K), jnp.float32),
        ],
    )(out, vals)


def _sc_body(sh_hbm, y_hbm, tk_hbm, o_hbm, sp_hbm,
             s1_out, h_out, cs_out, cm_out,
             mask_v, sh_v0, y_v0, tk_v0, o_v0, sp_v0,
             sh_v1, y_v1, tk_v1, o_v1, sp_v1,
             r1_v, r2_v, r3_v, r4_v, sem0, sem1):
    wid = lax.axis_index("s") * _NC + lax.axis_index("c")
    row0 = wid * _RPW
    neg1 = jnp.full((_LANES,), -1, jnp.int32)
    bufs = ((sh_v0, y_v0, tk_v0, o_v0, sp_v0),
            (sh_v1, y_v1, tk_v1, o_v1, sp_v1))
    sems = (sem0, sem1)

    def _copies(ci, slot):
        base = row0 + ci * _CH
        sh_v, y_v, tk_v, o_v, sp_v = bufs[slot]
        return (
            (sh_hbm.at[pl.ds(base * _S, _CH * _S)], sh_v.at[pl.ds(0, _CH * _S)]),
            (y_hbm.at[pl.ds(base * _LY, _CH * _LY)], y_v.at[pl.ds(0, _CH * _LY)]),
            (tk_hbm.at[pl.ds(base * _K, _CH * _K)], tk_v.at[pl.ds(0, _CH * _K)]),
            (o_hbm.at[pl.ds(base * _S, _CH * _S)], o_v.at[pl.ds(0, _CH * _S)]),
            (sp_hbm.at[pl.ds(base * _K, _CH * _K)], sp_v.at[pl.ds(0, _CH * _K)]),
        )

    def _issue(ci, slot):
        for src, dst in _copies(ci, slot):
            pltpu.async_copy(src, dst, sems[slot])

    def _wait(ci, slot):
        for src, dst in _copies(ci, slot):
            pltpu.make_async_copy(src, dst, sems[slot]).wait()

    _issue(0, 0)
    _issue(1, 1)

    zi = jnp.zeros((_LANES,), jnp.int32)
    zf = jnp.zeros((_LANES,), jnp.float32)
    for sh_v, y_v, tk_v, o_v, sp_v in bufs:
        sh_v[pl.ds(_CH * _S, 16)] = zi
        y_v[pl.ds(_CH * _LY, 16)] = zi
        tk_v[pl.ds(_CH * _K, 16)] = zi
        o_v[pl.ds(_CH * _S, 16)] = zf
        sp_v[pl.ds(_CH * _K, 16)] = zf

    def init_body(i, _):
        for t in range(16):
            mask_v[pl.ds(i * 256 + t * 16, 16)] = neg1
        return 0

    lax.fori_loop(0, _MASKN // 256, init_body, 0)

    onei = jnp.full((_LANES,), 1, jnp.int32)
    iota = lax.iota(jnp.int32, _LANES)
    padq = jnp.full((_LANES,), _PADQ, jnp.int32)
    pady = jnp.full((_LANES,), _PADY, jnp.int32)
    m_y = iota < (_LY % 16)
    m_sh = iota < (_S % 16)
    m_tk = iota < (_K % 16)
    n_y, n_sh, n_tk = _LY // 16, _S // 16, _K // 16
    last = _NCHUNK - 1

    def _compute(ci, slot, carry):
        s1, h, csum, cmax = carry
        base = row0 + ci * _CH
        sh_v, y_v, tk_v, o_v, sp_v = bufs[slot]
        for r in range(_CH):
            tag = jnp.full((_LANES,), base + r, jnp.int32)
            for t in range(n_y + 1):
                yv = y_v[pl.ds(r * _LY + t * 16, 16)]
                if t == n_y:
                    yv = jnp.where(m_y, yv, pady)
                plsc.store_scatter(mask_v, [yv], tag)
            for j in range(n_sh + 1):
                q = sh_v[pl.ds(r * _S + j * 16, 16)]
                if j == n_sh:
                    q = jnp.where(m_sh, q, padq)
                m = plsc.load_gather(mask_v, [q])
                o = o_v[pl.ds(r * _S + j * 16, 16)]
                s1 = s1 + jnp.where(m == tag, o, zf)
            rowcnt = zi
            for j in range(n_tk + 1):
                q = tk_v[pl.ds(r * _K + j * 16, 16)]
                if j == n_tk:
                    q = jnp.where(m_tk, q, padq)
                m = plsc.load_gather(mask_v, [q])
                spv = sp_v[pl.ds(r * _K + j * 16, 16)]
                hit = m == tag
                h = h + jnp.where(hit, spv, zf)
                rowcnt = rowcnt + jnp.where(hit, onei, zi)
            cmax = jnp.maximum(cmax, jnp.sum(rowcnt))
            csum = csum + rowcnt
        return (s1, h, csum, cmax)

    def pair_body(p, carry):
        c0 = p * 2
        c1 = c0 + 1
        _wait(c0, 0)
        carry = _compute(c0, 0, carry)
        _issue(jnp.minimum(c0 + 2, last), 0)
        _wait(c1, 1)
        carry = _compute(c1, 1, carry)
        _issue(jnp.minimum(c1 + 2, last), 1)
        return carry

    s1, h, csum, cmax = lax.fori_loop(
        0, _NCHUNK // 2, pair_body,
        (zf, zf, zi, jnp.int32(0)))
    _wait(last, 0)
    _wait(last, 1)

    r1_v[...] = s1
    r2_v[...] = h
    r3_v[...] = csum
    r4_v[...] = jnp.full((_LANES,), cmax, jnp.int32)
    pltpu.sync_copy(r1_v, s1_out.at[pl.ds(wid * _LANES, _LANES)])
    pltpu.sync_copy(r2_v, h_out.at[pl.ds(wid * _LANES, _LANES)])
    pltpu.sync_copy(r3_v, cs_out.at[pl.ds(wid * _LANES, _LANES)])
    pltpu.sync_copy(r4_v, cm_out.at[pl.ds(wid * _LANES, _LANES)])


def _sc_membership(sh_p, y_p, tk_p, out_p, sp_p):
    mesh = plsc.VectorSubcoreMesh(core_axis_name="c", subcore_axis_name="s")
    buf_pair = [
        pltpu.VMEM((_CH * _S + 16,), jnp.int32),
        pltpu.VMEM((_CH * _LY + 16,), jnp.int32),
        pltpu.VMEM((_CH * _K + 16,), jnp.int32),
        pltpu.VMEM((_CH * _S + 16,), jnp.float32),
        pltpu.VMEM((_CH * _K + 16,), jnp.float32),
    ]
    f = pl.kernel(
        _sc_body,
        out_type=[
            jax.ShapeDtypeStruct((_NW * _LANES,), jnp.float32),
            jax.ShapeDtypeStruct((_NW * _LANES,), jnp.float32),
            jax.ShapeDtypeStruct((_NW * _LANES,), jnp.int32),
            jax.ShapeDtypeStruct((_NW * _LANES,), jnp.int32),
        ],
        mesh=mesh,
        compiler_params=pltpu.CompilerParams(needs_layout_passes=False),
        scratch_types=(
            [pltpu.VMEM((_MASKN,), jnp.int32)]
            + buf_pair + buf_pair
            + [
                pltpu.VMEM((_LANES,), jnp.float32),
                pltpu.VMEM((_LANES,), jnp.float32),
                pltpu.VMEM((_LANES,), jnp.int32),
                pltpu.VMEM((_LANES,), jnp.int32),
                pltpu.SemaphoreType.DMA,
                pltpu.SemaphoreType.DMA,
            ]
        ),
    )
    return f(sh_p, y_p, tk_p, out_p, sp_p)


def kernel(out, shorty, topk_C_vals, topk_C_inds, y_inds):
    B, S = out.shape
    sh_f = shorty.astype(jnp.int32).reshape(-1)
    tk_f = topk_C_inds.astype(jnp.int32).reshape(-1)
    y_f = y_inds.astype(jnp.int32).reshape(-1)
    o_f = out.reshape(-1)

    a_arr, sp = _tc_dense(out, topk_C_vals)
    s1_w, h_w, cs_w, cm_w = _sc_membership(sh_f, y_f, tk_f, o_f,
                                           sp.reshape(-1))

    A = a_arr[0, 0]
    S1 = jnp.sum(s1_w)
    H = jnp.sum(h_w)
    csum = jnp.sum(cs_w).astype(jnp.float32)
    k = jnp.maximum(jnp.max(cm_w), 1).astype(jnp.float32)
    loss_precision = (A - S1) / jnp.float32(B * S)
    n = jnp.float32(B) * k
    loss_recall = (H + (n - csum) * jnp.float32(LOG2_)) / n
    return loss_precision + jnp.float32(GAMMA_) * loss_recall


# speedup vs baseline: 160.3889x; 1.0061x over previous
import functools

import jax
import jax.numpy as jnp
from jax import lax
from jax.experimental import pallas as pl
from jax.experimental.pallas import tpu as pltpu
from jax.experimental.pallas import tpu_sc as plsc

GAMMA_ = 0.05
LOG2_ = 0.6931471805599453

_NC, _NS, _LANES = 2, 16, 16
_NW = _NC * _NS

_B, _S, _K, _LY = 4096, 500, 200, 50
_PADQ = 100001
_PADY = 100000
_MASKN = 100352

_RPW = _B // _NW
_CH = 4
_NCHUNK = _RPW // _CH


def _tc_sp_body(vals_ref, sp_ref):
    v = vals_ref[...]
    sp_ref[...] = jnp.maximum(v, 0.0) - v + jnp.log(1.0 + jnp.exp(-jnp.abs(v)))


def _tc_sp(vals_f):
    n = _B * _K
    blk = n // 8
    return pl.pallas_call(
        _tc_sp_body,
        grid=(8,),
        in_specs=[pl.BlockSpec((blk,), lambda i: (i,))],
        out_specs=pl.BlockSpec((blk,), lambda i: (i,)),
        out_shape=jax.ShapeDtypeStruct((n,), jnp.float32),
    )(vals_f)


def _tc_a_body(out_ref, a_ref):
    step = pl.program_id(0)
    x = out_ref[...]
    a_part = jnp.sum(jnp.maximum(x, 0.0) + jnp.log(1.0 + jnp.exp(-jnp.abs(x))))

    @pl.when(step == 0)
    def _():
        a_ref[0, 0] = 0.0

    a_ref[0, 0] += a_part


def _tc_a(out_f):
    n = _B * _S
    blk = n // 8
    return pl.pallas_call(
        _tc_a_body,
        grid=(8,),
        in_specs=[pl.BlockSpec((blk,), lambda i: (i,))],
        out_specs=pl.BlockSpec(memory_space=pltpu.SMEM),
        out_shape=jax.ShapeDtypeStruct((1, 1), jnp.float32),
    )(out_f)


def _sc_body(sh_hbm, y_hbm, tk_hbm, o_hbm, sp_hbm,
             res_out,
             mask_v, sh_v0, y_v0, tk_v0, o_v0, sp_v0,
             sh_v1, y_v1, tk_v1, o_v1, sp_v1,
             r1_v, r2_v, r3_v, r4_v, sem0, sem1):
    wid = lax.axis_index("s") * _NC + lax.axis_index("c")
    row0 = wid * _RPW
    neg1 = jnp.full((_LANES,), -1, jnp.int32)
    bufs = ((sh_v0, y_v0, tk_v0, o_v0, sp_v0),
            (sh_v1, y_v1, tk_v1, o_v1, sp_v1))
    sems = (sem0, sem1)

    def _copies(ci, slot):
        base = row0 + ci * _CH
        sh_v, y_v, tk_v, o_v, sp_v = bufs[slot]
        return (
            (sh_hbm.at[pl.ds(base * _S, _CH * _S)], sh_v.at[pl.ds(0, _CH * _S)]),
            (y_hbm.at[pl.ds(base * _LY, _CH * _LY)], y_v.at[pl.ds(0, _CH * _LY)]),
            (tk_hbm.at[pl.ds(base * _K, _CH * _K)], tk_v.at[pl.ds(0, _CH * _K)]),
            (o_hbm.at[pl.ds(base * _S, _CH * _S)], o_v.at[pl.ds(0, _CH * _S)]),
            (sp_hbm.at[pl.ds(base * _K, _CH * _K)], sp_v.at[pl.ds(0, _CH * _K)]),
        )

    def _issue(ci, slot):
        for src, dst in _copies(ci, slot):
            pltpu.async_copy(src, dst, sems[slot])

    def _wait(ci, slot):
        for src, dst in _copies(ci, slot):
            pltpu.make_async_copy(src, dst, sems[slot]).wait()

    _issue(0, 0)
    _issue(1, 1)

    zi = jnp.zeros((_LANES,), jnp.int32)
    zf = jnp.zeros((_LANES,), jnp.float32)
    for sh_v, y_v, tk_v, o_v, sp_v in bufs:
        sh_v[pl.ds(_CH * _S, 16)] = zi
        y_v[pl.ds(_CH * _LY, 16)] = zi
        tk_v[pl.ds(_CH * _K, 16)] = zi
        o_v[pl.ds(_CH * _S, 16)] = zf
        sp_v[pl.ds(_CH * _K, 16)] = zf

    def init_body(i, _):
        for t in range(16):
            mask_v[pl.ds(i * 256 + t * 16, 16)] = neg1
        return 0

    lax.fori_loop(0, _MASKN // 256, init_body, 0)

    onei = jnp.full((_LANES,), 1, jnp.int32)
    iota = lax.iota(jnp.int32, _LANES)
    padq = jnp.full((_LANES,), _PADQ, jnp.int32)
    pady = jnp.full((_LANES,), _PADY, jnp.int32)
    m_y = iota < (_LY % 16)
    m_sh = iota < (_S % 16)
    m_tk = iota < (_K % 16)
    n_y, n_sh, n_tk = _LY // 16, _S // 16, _K // 16
    last = _NCHUNK - 1

    def _compute(ci, slot, carry):
        s1, h, csum, cmax = carry
        base = row0 + ci * _CH
        sh_v, y_v, tk_v, o_v, sp_v = bufs[slot]
        for r in range(_CH):
            tag = jnp.full((_LANES,), base + r, jnp.int32)
            for t in range(n_y + 1):
                yv = y_v[pl.ds(r * _LY + t * 16, 16)]
                if t == n_y:
                    yv = jnp.where(m_y, yv, pady)
                plsc.store_scatter(mask_v, [yv], tag)
            for j in range(n_sh + 1):
                q = sh_v[pl.ds(r * _S + j * 16, 16)]
                if j == n_sh:
                    q = jnp.where(m_sh, q, padq)
                m = plsc.load_gather(mask_v, [q])
                o = o_v[pl.ds(r * _S + j * 16, 16)]
                s1 = s1 + jnp.where(m == tag, o, zf)
            rowcnt = zi
            for j in range(n_tk + 1):
                q = tk_v[pl.ds(r * _K + j * 16, 16)]
                if j == n_tk:
                    q = jnp.where(m_tk, q, padq)
                m = plsc.load_gather(mask_v, [q])
                spv = sp_v[pl.ds(r * _K + j * 16, 16)]
                hit = m == tag
                h = h + jnp.where(hit, spv, zf)
                rowcnt = rowcnt + jnp.where(hit, onei, zi)
            cmax = jnp.maximum(cmax, jnp.sum(rowcnt))
            csum = csum + rowcnt
        return (s1, h, csum, cmax)

    def pair_body(p, carry):
        c0 = p * 2
        c1 = c0 + 1
        _wait(c0, 0)
        carry = _compute(c0, 0, carry)
        _issue(jnp.minimum(c0 + 2, last), 0)
        _wait(c1, 1)
        carry = _compute(c1, 1, carry)
        _issue(jnp.minimum(c1 + 2, last), 1)
        return carry

    s1, h, csum, cmax = lax.fori_loop(
        0, _NCHUNK // 2, pair_body,
        (zf, zf, zi, jnp.int32(0)))
    _wait(last, 0)
    _wait(last, 1)

    nwl = _NW * _LANES
    r1_v[...] = s1
    r2_v[...] = h
    r3_v[...] = csum.astype(jnp.float32)
    r4_v[...] = jnp.full((_LANES,), cmax, jnp.int32).astype(jnp.float32)
    pltpu.sync_copy(r1_v, res_out.at[pl.ds(wid * _LANES, _LANES)])
    pltpu.sync_copy(r2_v, res_out.at[pl.ds(nwl + wid * _LANES, _LANES)])
    pltpu.sync_copy(r3_v, res_out.at[pl.ds(2 * nwl + wid * _LANES, _LANES)])
    pltpu.sync_copy(r4_v, res_out.at[pl.ds(3 * nwl + wid * _LANES, _LANES)])


def _sc_membership(sh_p, y_p, tk_p, out_p, sp_p):
    mesh = plsc.VectorSubcoreMesh(core_axis_name="c", subcore_axis_name="s")
    buf_pair = [
        pltpu.VMEM((_CH * _S + 16,), jnp.int32),
        pltpu.VMEM((_CH * _LY + 16,), jnp.int32),
        pltpu.VMEM((_CH * _K + 16,), jnp.int32),
        pltpu.VMEM((_CH * _S + 16,), jnp.float32),
        pltpu.VMEM((_CH * _K + 16,), jnp.float32),
    ]
    f = pl.kernel(
        _sc_body,
        out_type=jax.ShapeDtypeStruct((4 * _NW * _LANES,), jnp.float32),
        mesh=mesh,
        compiler_params=pltpu.CompilerParams(needs_layout_passes=False),
        scratch_types=(
            [pltpu.VMEM((_MASKN,), jnp.int32)]
            + buf_pair + buf_pair
            + [
                pltpu.VMEM((_LANES,), jnp.float32),
                pltpu.VMEM((_LANES,), jnp.float32),
                pltpu.VMEM((_LANES,), jnp.float32),
                pltpu.VMEM((_LANES,), jnp.float32),
                pltpu.SemaphoreType.DMA,
                pltpu.SemaphoreType.DMA,
            ]
        ),
    )
    return f(sh_p, y_p, tk_p, out_p, sp_p)


def kernel(out, shorty, topk_C_vals, topk_C_inds, y_inds):
    B, S = out.shape
    sh_f = shorty.astype(jnp.int32).reshape(-1)
    tk_f = topk_C_inds.astype(jnp.int32).reshape(-1)
    y_f = y_inds.astype(jnp.int32).reshape(-1)
    o_f = out.reshape(-1)

    sp_f = _tc_sp(topk_C_vals.reshape(-1))
    res = _sc_membership(sh_f, y_f, tk_f, o_f, sp_f)
    a_arr = _tc_a(o_f)

    nwl = _NW * _LANES
    A = a_arr[0, 0]
    S1 = jnp.sum(res[:nwl])
    H = jnp.sum(res[nwl:2 * nwl])
    csum = jnp.sum(res[2 * nwl:3 * nwl])
    k = jnp.maximum(jnp.max(res[3 * nwl:]), 1.0)
    loss_precision = (A - S1) / jnp.float32(B * S)
    n = jnp.float32(B) * k
    loss_recall = (H + (n - csum) * jnp.float32(LOG2_)) / n
    return loss_precision + jnp.float32(GAMMA_) * loss_recall
